# baseline (speedup 1.0000x reference)
import functools

import jax
import jax.numpy as jnp
from jax import lax
from jax.experimental import pallas as pl
from jax.experimental.pallas import tpu as pltpu
from jax.experimental.pallas import tpu_sc as plsc

_W = 224
_C = 384
_D = 192
_WG = 4
_PG = 8
_WROWS = _W // _WG
_SLAB = _WROWS * _C
_TROWS = _WROWS * _D
_NBUF = 2


def _sc_body(x_hbm, row_hbm, col_hbm, out_hbm,
             eb, ib0, ib1, ob0, ob1,
             gs0, gs1, ss0, ss1, *, nc, periods):
    wid = lax.axis_index("s") * nc + lax.axis_index("c")
    wg = lax.rem(wid, _WG)
    pg = lax.div(wid, _WG)
    ibufs = (ib0, ib1)
    obufs = (ob0, ob1)
    gsems = (gs0, gs1)
    ssems = (ss0, ss1)

    tab_off = pl.multiple_of(wg * _TROWS, 8)
    pltpu.sync_copy(row_hbm.at[pl.ds(tab_off, _TROWS)], ib0.at[pl.ds(0, _TROWS)])
    pltpu.sync_copy(col_hbm.at[pl.ds(tab_off, _TROWS)],
                    ib0.at[pl.ds(_TROWS, _TROWS)])

    def interleave_row(r, c2):
        xo = pl.multiple_of(r * _C, 8)
        to = pl.multiple_of(r * _D, 8)
        for j in range(_D // 16):
            eb[pl.ds(xo + 16 * j, 16)] = ib0[pl.ds(to + 16 * j, 16)]
        for j in range(_D // 16):
            eb[pl.ds(xo + _D + 16 * j, 16)] = ib0[pl.ds(_TROWS + to + 16 * j, 16)]
        return c2

    lax.fori_loop(0, _WROWS, interleave_row, 0)

    p0 = pg * periods
    row0 = wg * _WROWS

    def slab_off(p_local):
        return pl.multiple_of(((p0 + p_local) * _W + row0) * _C, 8)

    def start_gather(s, p_local):
        pltpu.make_async_copy(
            x_hbm.at[pl.ds(slab_off(p_local), _SLAB)], ibufs[s], gsems[s]
        ).start()

    def wait_gather(s):
        pltpu.make_async_copy(
            x_hbm.at[pl.ds(0, _SLAB)], ibufs[s], gsems[s]
        ).wait()

    def start_scatter(s, p_local):
        pltpu.make_async_copy(
            obufs[s], out_hbm.at[pl.ds(slab_off(p_local), _SLAB)], ssems[s]
        ).start()

    def wait_scatter(s):
        pltpu.make_async_copy(
            obufs[s], out_hbm.at[pl.ds(0, _SLAB)], ssems[s]
        ).wait()

    for s in range(_NBUF):
        start_gather(s, s)

    def outer(g, carry):
        for s in range(_NBUF):
            p_local = g * _NBUF + s
            wait_gather(s)

            @pl.when(g >= 1)
            def _():
                wait_scatter(s)

            def add_chunk(k, c2, ib=ibufs[s], ob=obufs[s]):
                base = pl.multiple_of(k * 128, 8)
                for u in range(8):
                    o = base + 16 * u
                    ob[pl.ds(o, 16)] = ib[pl.ds(o, 16)] + eb[pl.ds(o, 16)]
                return c2

            lax.fori_loop(0, _SLAB // 128, add_chunk, 0)
            start_scatter(s, p_local)

            @pl.when(g < periods // _NBUF - 1)
            def _():
                start_gather(s, p_local + _NBUF)
        return carry

    lax.fori_loop(0, periods // _NBUF, outer, 0)
    for s in range(_NBUF):
        wait_scatter(s)


def kernel(inputs, row_table, col_table):
    B, H, W, C = inputs.shape
    periods = B * H // _PG
    info = plsc.get_sparse_core_info()
    nc = info.num_cores
    mesh = plsc.VectorSubcoreMesh(core_axis_name="c", subcore_axis_name="s")
    n = B * H * W * C
    k = functools.partial(
        pl.kernel,
        mesh=mesh,
        out_type=jax.ShapeDtypeStruct((n,), jnp.float32),
        scratch_types=(
            [pltpu.VMEM((_SLAB,), jnp.float32)] * (2 * _NBUF + 1)
            + [pltpu.SemaphoreType.DMA] * (2 * _NBUF)
        ),
    )(functools.partial(_sc_body, nc=nc, periods=periods))
    out = k(inputs.reshape(-1), row_table.reshape(-1), col_table.reshape(-1))
    return out.reshape(B, H, W, C)


# speedup vs baseline: 4.3808x; 4.3808x over previous
import functools

import jax
import jax.numpy as jnp
from jax.experimental import pallas as pl
from jax.experimental.pallas import tpu as pltpu


def _add_body(x_ref, row_ref, col_ref, o_ref, emb_ref, *, d):
    @pl.when(pl.program_id(0) == 0)
    def _():
        emb_ref[:, :d] = row_ref[...]
        emb_ref[:, d:] = col_ref[...]

    o_ref[...] = x_ref[...] + emb_ref[...][None, :, :]


def kernel(inputs, row_table, col_table):
    B, H, W, C = inputs.shape
    d = row_table.shape[1]
    K = 28
    x = inputs.reshape(B * H, W, C)
    grid = (B * H // K,)
    out = pl.pallas_call(
        functools.partial(_add_body, d=d),
        grid=grid,
        in_specs=[
            pl.BlockSpec((K, W, C), lambda i: (i, 0, 0)),
            pl.BlockSpec((W, d), lambda i: (0, 0)),
            pl.BlockSpec((H, d), lambda i: (0, 0)),
        ],
        out_specs=pl.BlockSpec((K, W, C), lambda i: (i, 0, 0)),
        out_shape=jax.ShapeDtypeStruct((B * H, W, C), inputs.dtype),
        scratch_shapes=[pltpu.VMEM((W, C), inputs.dtype)],
    )(x, row_table, col_table)
    return out.reshape(B, H, W, C)


# speedup vs baseline: 4.3934x; 1.0029x over previous
import functools

import jax
import jax.numpy as jnp
from jax.experimental import pallas as pl
from jax.experimental.pallas import tpu as pltpu


def _add_body(x_ref, row_ref, col_ref, o_ref, emb_ref, *, d):
    @pl.when(pl.program_id(0) == 0)
    def _():
        emb_ref[:, :d] = row_ref[...]
        emb_ref[:, d:] = col_ref[...]

    o_ref[...] = x_ref[...] + emb_ref[...][None, :, :]


def kernel(inputs, row_table, col_table):
    B, H, W, C = inputs.shape
    d = row_table.shape[1]
    K = 32
    x = inputs.reshape(B * H, W, C)
    grid = (B * H // K,)
    out = pl.pallas_call(
        functools.partial(_add_body, d=d),
        grid=grid,
        in_specs=[
            pl.BlockSpec((K, W, C), lambda i: (i, 0, 0)),
            pl.BlockSpec((W, d), lambda i: (0, 0)),
            pl.BlockSpec((H, d), lambda i: (0, 0)),
        ],
        out_specs=pl.BlockSpec((K, W, C), lambda i: (i, 0, 0)),
        out_shape=jax.ShapeDtypeStruct((B * H, W, C), inputs.dtype),
        scratch_shapes=[pltpu.VMEM((W, C), inputs.dtype)],
    )(x, row_table, col_table)
    return out.reshape(B, H, W, C)
